# Initial kernel scaffold; baseline (speedup 1.0000x reference)
import jax
import jax.numpy as jnp
from jax.experimental import pallas as pl


def kernel(U, w_squeeze, w_excitation):
    raise NotImplementedError("write your pallas kernel here")


# speedup vs baseline: 2.3694x; 2.3694x over previous
import functools

import jax
import jax.numpy as jnp
from jax.experimental import pallas as pl
from jax.experimental.pallas import tpu as pltpu


def _se_kernel(u_ref, w1_ref, w2_ref, o_ref, *, inv_n):
    u = u_ref[0].astype(jnp.float32)
    z = (jnp.sum(u, axis=1, keepdims=True) * inv_n
         + jnp.max(u, axis=1, keepdims=True))
    h = jnp.dot(w1_ref[...], z, preferred_element_type=jnp.float32)
    s = jnp.dot(w2_ref[...], h, preferred_element_type=jnp.float32)
    gate = jax.nn.sigmoid(s)
    o_ref[0] = jnp.broadcast_to(gate.astype(o_ref.dtype), o_ref.shape[1:])


def kernel(U, w_squeeze, w_excitation):
    B, C, H, W = U.shape
    N = H * W
    u3d = U.reshape(B, C, N)
    w1 = w_squeeze.reshape(C // 2, C).astype(jnp.float32)
    w2 = w_excitation.reshape(C, C // 2).astype(jnp.float32)

    out = pl.pallas_call(
        functools.partial(_se_kernel, inv_n=1.0 / N),
        out_shape=jax.ShapeDtypeStruct((B, C, N), U.dtype),
        grid=(B,),
        in_specs=[
            pl.BlockSpec((1, C, N), lambda b: (b, 0, 0)),
            pl.BlockSpec((C // 2, C), lambda b: (0, 0)),
            pl.BlockSpec((C, C // 2), lambda b: (0, 0)),
        ],
        out_specs=pl.BlockSpec((1, C, N), lambda b: (b, 0, 0)),
        compiler_params=pltpu.CompilerParams(
            dimension_semantics=("parallel",)),
    )(u3d, w1, w2)
    return out.reshape(B, C, H, W)


# speedup vs baseline: 2.4934x; 1.0523x over previous
import functools

import jax
import jax.numpy as jnp
from jax.experimental import pallas as pl
from jax.experimental.pallas import tpu as pltpu


def _se_kernel(u_ref, w1_ref, w2_ref, o_ref, *, inv_n, bb):
    for b in range(bb):
        u = u_ref[b].astype(jnp.float32)
        z = (jnp.sum(u, axis=1, keepdims=True) * inv_n
             + jnp.max(u, axis=1, keepdims=True))
        h = jnp.dot(w1_ref[...], z, preferred_element_type=jnp.float32)
        s = jnp.dot(w2_ref[...], h, preferred_element_type=jnp.float32)
        gate = jax.nn.sigmoid(s)
        o_ref[b] = jnp.broadcast_to(gate.astype(o_ref.dtype),
                                    o_ref.shape[1:])


def kernel(U, w_squeeze, w_excitation):
    B, C, H, W = U.shape
    N = H * W
    bb = 2 if B % 2 == 0 else 1
    u3d = U.reshape(B, C, N)
    w1 = w_squeeze.reshape(C // 2, C).astype(jnp.float32)
    w2 = w_excitation.reshape(C, C // 2).astype(jnp.float32)

    out = pl.pallas_call(
        functools.partial(_se_kernel, inv_n=1.0 / N, bb=bb),
        out_shape=jax.ShapeDtypeStruct((B, C, N), U.dtype),
        grid=(B // bb,),
        in_specs=[
            pl.BlockSpec((bb, C, N), lambda b: (b, 0, 0)),
            pl.BlockSpec((C // 2, C), lambda b: (0, 0)),
            pl.BlockSpec((C, C // 2), lambda b: (0, 0)),
        ],
        out_specs=pl.BlockSpec((bb, C, N), lambda b: (b, 0, 0)),
        compiler_params=pltpu.CompilerParams(
            dimension_semantics=("parallel",)),
    )(u3d, w1, w2)
    return out.reshape(B, C, H, W)
